# baseline (speedup 1.0000x reference)
import jax
import jax.numpy as jnp
from jax.experimental import pallas as pl
from jax.experimental.pallas import tpu as pltpu

_GH = 4


def _hwb_sum_kernel(x_ref, o_ref):
    b, gh, w, c = x_ref.shape
    q = gh * w
    z = jnp.dot(
        x_ref[...].reshape(b * q, c),
        jnp.ones((c, 128), jnp.float32),
        preferred_element_type=jnp.float32,
    )
    nq = q // 128
    zv = z.reshape(b, nq, 128, 128)
    row = jax.lax.broadcasted_iota(jnp.int32, (128, 128), 0)
    col = jax.lax.broadcasted_iota(jnp.int32, (128, 128), 1)
    m = (row == col).astype(jnp.float32)
    d = jnp.sum(zv * m[None, None], axis=2)
    o_ref[...] = d.transpose(1, 2, 0).reshape(gh, w, b)


def kernel(x):
    b, c, h, w = x.shape
    x4 = jnp.transpose(x, (0, 2, 3, 1))

    out_hwb = pl.pallas_call(
        _hwb_sum_kernel,
        out_shape=jax.ShapeDtypeStruct((h, w, b), x.dtype),
        grid=(h // _GH,),
        in_specs=[pl.BlockSpec((b, _GH, w, c), lambda j: (0, j, 0, 0))],
        out_specs=pl.BlockSpec((_GH, w, b), lambda j: (j, 0, 0)),
        compiler_params=pltpu.CompilerParams(
            dimension_semantics=("parallel",),
            vmem_limit_bytes=64 * 1024 * 1024,
        ),
    )(x4)
    return jnp.transpose(out_hwb, (2, 0, 1))


# speedup vs baseline: 1.2188x; 1.2188x over previous
import jax
import jax.numpy as jnp
from jax.experimental import pallas as pl
from jax.experimental.pallas import tpu as pltpu

_GH = 4


def _hwb_sum_kernel(x_ref, o_ref):
    b, gh, w, c = x_ref.shape
    q = gh * w
    z = jnp.dot(
        x_ref[...].reshape(b * q, c),
        jnp.ones((c, 128), jnp.float32),
        preferred_element_type=jnp.float32,
    )
    zv = z.reshape(b, q, 128)
    row = jax.lax.broadcasted_iota(jnp.int32, (q, 128), 0)
    col = jax.lax.broadcasted_iota(jnp.int32, (q, 128), 1)
    m = (row == col).astype(jnp.float32)
    d = jnp.sum(zv * m[None], axis=1)
    o_ref[...] = d.T.reshape(gh, w, b)


def kernel(x):
    b, c, h, w = x.shape
    x4 = jnp.transpose(x, (0, 2, 3, 1))

    out_hwb = pl.pallas_call(
        _hwb_sum_kernel,
        out_shape=jax.ShapeDtypeStruct((h, w, b), x.dtype),
        grid=(h // _GH,),
        in_specs=[pl.BlockSpec((b, _GH, w, c), lambda j: (0, j, 0, 0))],
        out_specs=pl.BlockSpec((_GH, w, b), lambda j: (j, 0, 0)),
        compiler_params=pltpu.CompilerParams(
            dimension_semantics=("parallel",),
            vmem_limit_bytes=64 * 1024 * 1024,
        ),
    )(x4)
    return jnp.transpose(out_hwb, (2, 0, 1))
